# baseline (speedup 1.0000x reference)
import jax
import jax.numpy as jnp
from jax.experimental import pallas as pl

T = 2048
D = 768
E = 16
FM = 256
G = 8
KG = 4
SCALE = 2.5
TT = 256

_DOT_PREC = jax.lax.Precision.DEFAULT


def _dot(a, b):
    return jax.lax.dot_general(a, b, (((1,), (1,)), ((), ())),
                               precision=_DOT_PREC,
                               preferred_element_type=jnp.float32)


def _moe_kernel(x_ref, xb_ref, gate_w_ref, gate_b_ref, w1_ref, w2_ref, w3_ref,
                sw1_ref, sw2_ref, sw3_ref, out_ref):
    x = x_ref[...]
    xb = xb_ref[...]

    scores = jax.nn.sigmoid(_dot(x, gate_w_ref[...]))
    sb = scores + gate_b_ref[...]
    gs = sb.reshape(TT, G, 2).sum(axis=-1)
    ga = gs[:, :, None]
    gb = gs[:, None, :]
    gidx = jax.lax.broadcasted_iota(jnp.int32, (TT, G, G), 1)
    oidx = jax.lax.broadcasted_iota(jnp.int32, (TT, G, G), 2)
    beats = jnp.logical_or(gb > ga, jnp.logical_and(gb == ga, oidx < gidx))
    rank = jnp.where(beats, 1.0, 0.0).sum(axis=-1)
    sel_g = jnp.where(rank < KG, 1.0, 0.0)
    sel_e = jnp.broadcast_to(sel_g[:, :, None], (TT, G, 2)).reshape(TT, E)
    w = sel_e * scores
    cw = w * (SCALE / w.sum(axis=-1, keepdims=True))

    hs = jax.nn.silu(_dot(xb, sw1_ref[...])) * _dot(xb, sw3_ref[...])
    acc = _dot(hs.astype(jnp.bfloat16), sw2_ref[...])

    for e in range(E):
        h1 = _dot(xb, w1_ref[e])
        h3 = _dot(xb, w3_ref[e])
        h = jax.nn.silu(h1) * h3 * cw[:, e:e + 1]
        acc += _dot(h.astype(jnp.bfloat16), w2_ref[e])

    out_ref[...] = acc


@jax.jit
def kernel(x, gate_w, gate_b, W1, W2, W3, sw1, sw2, sw3):
    grid = (T // TT,)
    return pl.pallas_call(
        _moe_kernel,
        grid=grid,
        in_specs=[
            pl.BlockSpec((TT, D), lambda t: (t, 0)),
            pl.BlockSpec((TT, D), lambda t: (t, 0)),
            pl.BlockSpec((E, D), lambda t: (0, 0)),
            pl.BlockSpec((1, E), lambda t: (0, 0)),
            pl.BlockSpec((E, FM, D), lambda t: (0, 0, 0)),
            pl.BlockSpec((E, D, FM), lambda t: (0, 0, 0)),
            pl.BlockSpec((E, FM, D), lambda t: (0, 0, 0)),
            pl.BlockSpec((FM, D), lambda t: (0, 0)),
            pl.BlockSpec((D, FM), lambda t: (0, 0)),
            pl.BlockSpec((FM, D), lambda t: (0, 0)),
        ],
        out_specs=pl.BlockSpec((TT, D), lambda t: (t, 0)),
        out_shape=jax.ShapeDtypeStruct((T, D), x.dtype),
    )(x, x.astype(jnp.bfloat16), gate_w, gate_b.reshape(1, E),
      W1.astype(jnp.bfloat16), W2.astype(jnp.bfloat16), W3.astype(jnp.bfloat16),
      sw1.astype(jnp.bfloat16), sw2.astype(jnp.bfloat16), sw3.astype(jnp.bfloat16))


# speedup vs baseline: 1.1451x; 1.1451x over previous
import jax
import jax.numpy as jnp
from jax.experimental import pallas as pl
from jax.experimental.pallas import tpu as pltpu

T = 2048
D = 768
E = 16
FM = 256
G = 8
KG = 4
SCALE = 2.5
TT = 256

_DOT_PREC = jax.lax.Precision.DEFAULT


def _dot(a, b):
    return jax.lax.dot_general(a, b, (((1,), (1,)), ((), ())),
                               precision=_DOT_PREC,
                               preferred_element_type=jnp.float32)


def _moe_kernel(x_ref, gate_w_ref, gate_b_ref, w1_ref, w2_ref, w3_ref,
                sw1_ref, sw2_ref, sw3_ref, out_ref, cw_ref):
    e = pl.program_id(0)

    @pl.when(e == 0)
    def _gate():
        x = x_ref[...]
        scores_t = jax.nn.sigmoid(_dot(gate_w_ref[...], x))
        sb_t = scores_t + gate_b_ref[...]
        gs_t = sb_t.reshape(G, 2, T).sum(axis=1)
        ga = gs_t[:, None, :]
        gb = gs_t[None, :, :]
        gidx = jax.lax.broadcasted_iota(jnp.int32, (G, G, T), 0)
        oidx = jax.lax.broadcasted_iota(jnp.int32, (G, G, T), 1)
        beats = jnp.logical_or(gb > ga,
                               jnp.logical_and(gb == ga, oidx < gidx))
        rank = jnp.where(beats, 1.0, 0.0).sum(axis=1)
        sel_g = jnp.where(rank < KG, 1.0, 0.0)
        sel_e = jnp.broadcast_to(sel_g[:, None, :], (G, 2, T)).reshape(E, T)
        w = sel_e * scores_t
        denom = w.sum(axis=0, keepdims=True)
        cw_ref[...] = (w * (SCALE / denom)).T

    onehot = jnp.where(
        jax.lax.broadcasted_iota(jnp.int32, (1, E), 1) == e, 1.0, 0.0)
    w1 = w1_ref[0]
    w3 = w3_ref[0]
    w2 = w2_ref[0]

    for i in range(T // TT):
        sl = pl.ds(i * TT, TT)
        x = x_ref[sl, :]
        cw_col = jnp.sum(cw_ref[sl, :] * onehot, axis=1, keepdims=True)
        h1 = _dot(x, w1)
        h3 = _dot(x, w3)
        h = jax.nn.silu(h1) * h3 * cw_col
        contrib = _dot(h, w2)

        @pl.when(e == 0)
        def _init():
            hs = jax.nn.silu(_dot(x, sw1_ref[...])) * _dot(x, sw3_ref[...])
            out_ref[sl, :] = _dot(hs, sw2_ref[...]) + contrib

        @pl.when(e != 0)
        def _accum():
            out_ref[sl, :] += contrib


@jax.jit
def kernel(x, gate_w, gate_b, W1, W2, W3, sw1, sw2, sw3):
    return pl.pallas_call(
        _moe_kernel,
        grid=(E,),
        in_specs=[
            pl.BlockSpec((T, D), lambda e: (0, 0)),
            pl.BlockSpec((E, D), lambda e: (0, 0)),
            pl.BlockSpec((E, 1), lambda e: (0, 0)),
            pl.BlockSpec((1, FM, D), lambda e: (e, 0, 0)),
            pl.BlockSpec((1, D, FM), lambda e: (e, 0, 0)),
            pl.BlockSpec((1, FM, D), lambda e: (e, 0, 0)),
            pl.BlockSpec((FM, D), lambda e: (0, 0)),
            pl.BlockSpec((D, FM), lambda e: (0, 0)),
            pl.BlockSpec((FM, D), lambda e: (0, 0)),
        ],
        out_specs=pl.BlockSpec((T, D), lambda e: (0, 0)),
        out_shape=jax.ShapeDtypeStruct((T, D), x.dtype),
        scratch_shapes=[pltpu.VMEM((T, E), jnp.float32)],
    )(x, gate_w, gate_b.reshape(E, 1), W1, W2, W3, sw1, sw2, sw3)


# speedup vs baseline: 1.7123x; 1.4953x over previous
import jax
import jax.numpy as jnp
from jax.experimental import pallas as pl
from jax.experimental.pallas import tpu as pltpu

T = 2048
D = 768
E = 16
FM = 256
G = 8
KG = 4
SCALE = 2.5
TT = 512

_DOT_PREC = jax.lax.Precision.DEFAULT


def _dot(a, b):
    return jax.lax.dot_general(a, b, (((1,), (1,)), ((), ())),
                               precision=_DOT_PREC,
                               preferred_element_type=jnp.float32)


def _moe_kernel(x_ref, gate_w_ref, gate_b_ref, w1_ref, w2_ref, w3_ref,
                sw1_ref, sw2_ref, sw3_ref, out_ref, cw_ref):
    t = pl.program_id(0)

    @pl.when(t == 0)
    def _gate():
        xall = x_ref[...]
        scores_t = jax.nn.sigmoid(_dot(gate_w_ref[...], xall))
        sb_t = scores_t + gate_b_ref[...]
        gs_t = sb_t.reshape(G, 2, T).sum(axis=1)
        ga = gs_t[:, None, :]
        gb = gs_t[None, :, :]
        gidx = jax.lax.broadcasted_iota(jnp.int32, (G, G, T), 0)
        oidx = jax.lax.broadcasted_iota(jnp.int32, (G, G, T), 1)
        beats = jnp.logical_or(gb > ga,
                               jnp.logical_and(gb == ga, oidx < gidx))
        rank = jnp.where(beats, 1.0, 0.0).sum(axis=1)
        sel_g = jnp.where(rank < KG, 1.0, 0.0)
        sel_e = jnp.broadcast_to(sel_g[:, None, :], (G, 2, T)).reshape(E, T)
        w = sel_e * scores_t
        denom = w.sum(axis=0, keepdims=True)
        cw_ref[...] = (w * (SCALE / denom)).T

    sl = pl.ds(t * TT, TT)
    x = x_ref[sl, :]
    cw = cw_ref[sl, :]

    hs = jax.nn.silu(_dot(x, sw1_ref[...])) * _dot(x, sw3_ref[...])
    acc = _dot(hs, sw2_ref[...])

    for e in range(E):
        h1 = _dot(x, w1_ref[e])
        h3 = _dot(x, w3_ref[e])
        h = jax.nn.silu(h1) * h3 * cw[:, e:e + 1]
        acc += _dot(h, w2_ref[e])

    out_ref[...] = acc


@jax.jit
def kernel(x, gate_w, gate_b, W1, W2, W3, sw1, sw2, sw3):
    return pl.pallas_call(
        _moe_kernel,
        grid=(T // TT,),
        in_specs=[
            pl.BlockSpec((T, D), lambda t: (0, 0)),
            pl.BlockSpec((E, D), lambda t: (0, 0)),
            pl.BlockSpec((E, 1), lambda t: (0, 0)),
            pl.BlockSpec((E, FM, D), lambda t: (0, 0, 0)),
            pl.BlockSpec((E, D, FM), lambda t: (0, 0, 0)),
            pl.BlockSpec((E, FM, D), lambda t: (0, 0, 0)),
            pl.BlockSpec((FM, D), lambda t: (0, 0)),
            pl.BlockSpec((D, FM), lambda t: (0, 0)),
            pl.BlockSpec((FM, D), lambda t: (0, 0)),
        ],
        out_specs=pl.BlockSpec((TT, D), lambda t: (t, 0)),
        out_shape=jax.ShapeDtypeStruct((T, D), x.dtype),
        scratch_shapes=[pltpu.VMEM((T, E), jnp.float32)],
    )(x, gate_w, gate_b.reshape(E, 1), W1, W2, W3, sw1, sw2, sw3)


# speedup vs baseline: 1.7370x; 1.0144x over previous
import jax
import jax.numpy as jnp
from jax.experimental import pallas as pl
from jax.experimental.pallas import tpu as pltpu

T = 2048
D = 768
E = 16
FM = 256
G = 8
KG = 4
SCALE = 2.5
TT = 512

_DOT_PREC = jax.lax.Precision.DEFAULT


def _dot(a, b):
    return jax.lax.dot_general(a, b, (((1,), (1,)), ((), ())),
                               precision=_DOT_PREC,
                               preferred_element_type=jnp.float32)


def _moe_kernel(x_ref, gate_w_ref, gate_b_ref, w1_ref, w2_ref, w3_ref,
                sw1_ref, sw2_ref, sw3_ref, out_ref):
    x = x_ref[...]

    scores_t = jax.nn.sigmoid(_dot(gate_w_ref[...], x))
    sb_t = scores_t + gate_b_ref[...]
    gs_t = sb_t.reshape(G, 2, TT).sum(axis=1)
    ga = gs_t[:, None, :]
    gb = gs_t[None, :, :]
    gidx = jax.lax.broadcasted_iota(jnp.int32, (G, G, TT), 0)
    oidx = jax.lax.broadcasted_iota(jnp.int32, (G, G, TT), 1)
    beats = jnp.logical_or(gb > ga,
                           jnp.logical_and(gb == ga, oidx < gidx))
    rank = jnp.where(beats, 1.0, 0.0).sum(axis=1)
    sel_g = jnp.where(rank < KG, 1.0, 0.0)
    sel_e = jnp.broadcast_to(sel_g[:, None, :], (G, 2, TT)).reshape(E, TT)
    w = sel_e * scores_t
    denom = w.sum(axis=0, keepdims=True)
    cw = (w * (SCALE / denom)).T

    hs = jax.nn.silu(_dot(x, sw1_ref[...])) * _dot(x, sw3_ref[...])
    acc = _dot(hs, sw2_ref[...])

    for e in range(E):
        h1 = _dot(x, w1_ref[e])
        h3 = _dot(x, w3_ref[e])
        h = jax.nn.silu(h1) * h3 * cw[:, e:e + 1]
        acc += _dot(h, w2_ref[e])

    out_ref[...] = acc


@jax.jit
def kernel(x, gate_w, gate_b, W1, W2, W3, sw1, sw2, sw3):
    return pl.pallas_call(
        _moe_kernel,
        grid=(T // TT,),
        in_specs=[
            pl.BlockSpec((TT, D), lambda t: (t, 0)),
            pl.BlockSpec((E, D), lambda t: (0, 0)),
            pl.BlockSpec((E, 1), lambda t: (0, 0)),
            pl.BlockSpec((E, FM, D), lambda t: (0, 0, 0)),
            pl.BlockSpec((E, D, FM), lambda t: (0, 0, 0)),
            pl.BlockSpec((E, FM, D), lambda t: (0, 0, 0)),
            pl.BlockSpec((FM, D), lambda t: (0, 0)),
            pl.BlockSpec((D, FM), lambda t: (0, 0)),
            pl.BlockSpec((FM, D), lambda t: (0, 0)),
        ],
        out_specs=pl.BlockSpec((TT, D), lambda t: (t, 0)),
        out_shape=jax.ShapeDtypeStruct((T, D), x.dtype),
        compiler_params=pltpu.CompilerParams(
            dimension_semantics=("parallel",)),
    )(x, gate_w, gate_b.reshape(E, 1), W1, W2, W3, sw1, sw2, sw3)
